# baseline (speedup 1.0000x reference)
import jax
import jax.numpy as jnp
from jax.experimental import pallas as pl
from jax.experimental.pallas import tpu as pltpu

NEG = -3.0e38
K = 20
BLK_B = 8
W = 256
S_FAST = 5


def _insert_topk(state, x):
    new_state = []
    cur = x
    for s in state:
        hi = jnp.maximum(s, cur)
        cur = jnp.minimum(s, cur)
        new_state.append(hi)
    return new_state


def _extract_topk(cand, k):
    r, c = cand.shape
    lane = jax.lax.broadcasted_iota(jnp.int32, (r, c), 1)
    big = jnp.int32(2**30)
    work = cand
    outs = []
    for _ in range(k):
        g = jnp.max(work, axis=1, keepdims=True)
        outs.append(g)
        eq = work == g
        idx = jnp.min(jnp.where(eq, lane, big), axis=1, keepdims=True)
        work = jnp.where(lane == idx, NEG, work)
    return jnp.concatenate(outs, axis=1)


def _merge_sorted(a, b, out_len):
    la, lb = len(a), len(b)
    out = []
    for i in range(out_len):
        terms = []
        if i < la:
            terms.append(a[i])
        if i < lb:
            terms.append(b[i])
        for j in range(i):
            kk = i - 1 - j
            if j < la and kk < lb:
                terms.append(jnp.minimum(a[j], b[kk]))
        while len(terms) > 1:
            terms = [jnp.maximum(terms[t], terms[t + 1])
                     for t in range(0, len(terms) - 1, 2)] + (
                         [terms[-1]] if len(terms) % 2 else [])
        out.append(terms[0])
    return out


def _tree_topk(state, k):
    w = state[0].shape[1]
    while w > 1:
        half = w // 2
        a = [s[:, :half] for s in state]
        b = [s[:, half:] for s in state]
        state = _merge_sorted(a, b, min(2 * len(state), k))
        w = half
    return jnp.concatenate(state, axis=1)


def _phase1_body(x_ref, out_ref):
    v = x_ref.shape[1]
    nfull = v // W
    rem = v % W

    def load(c):
        return x_ref[:, pl.ds(c * W, W)]

    def load_rem():
        xr = x_ref[:, pl.ds(nfull * W, rem)]
        pad = jnp.full((BLK_B, W - rem), NEG, jnp.float32)
        return jnp.concatenate([xr, pad], axis=1)

    state0 = [jnp.full((BLK_B, W), NEG, jnp.float32) for _ in range(S_FAST)]
    ngroup = nfull // 4

    def sort4(a, b, c, d):
        p = [jnp.maximum(a, b), jnp.minimum(a, b)]
        q = [jnp.maximum(c, d), jnp.minimum(c, d)]
        return _merge_sorted(p, q, 4)

    def body1(t, state):
        g = sort4(load(4 * t), load(4 * t + 1),
                  load(4 * t + 2), load(4 * t + 3))
        return _merge_sorted(state, g, S_FAST)

    state = jax.lax.fori_loop(0, ngroup, body1, state0)
    tail = [load(c) for c in range(4 * ngroup, nfull)]
    if rem:
        tail.append(load_rem())
    for x in tail:
        state = _merge_sorted(state, [x], S_FAST)

    m = jnp.max(state[0], axis=1, keepdims=True)

    topk = _tree_topk(state, K)
    tau = topk[:, K - 1:K]
    bad = jnp.any(state[S_FAST - 1] >= tau)

    def acc_one(x, carry):
        s_acc, t_acc = carry
        e = jnp.exp(x - m)
        return s_acc + e, t_acc + e * x

    def body2(t, carry):
        return tuple(acc_one(load(4 * t + u), carry[u]) for u in range(4))

    zero = jnp.zeros((BLK_B, W), jnp.float32)
    accs = jax.lax.fori_loop(0, ngroup, body2, ((zero, zero),) * 4)
    accs = list(accs)
    for u, c in enumerate(range(4 * ngroup, nfull)):
        accs[u] = acc_one(load(c), accs[u])
    if rem:
        accs[3] = acc_one(load_rem(), accs[3])

    s_tot = (accs[0][0] + accs[1][0]) + (accs[2][0] + accs[3][0])
    t_tot = (accs[0][1] + accs[1][1]) + (accs[2][1] + accs[3][1])
    s = jnp.sum(s_tot, axis=1, keepdims=True)
    t = jnp.sum(t_tot, axis=1, keepdims=True)
    lse = m + jnp.log(s)
    entropy = lse - t / s

    out_ref[...] = jnp.concatenate([topk, lse, entropy], axis=1)

    @pl.when(bad)
    def _exact_fallback():
        st0 = [jnp.full((BLK_B, W), NEG, jnp.float32) for _ in range(K)]
        st = jax.lax.fori_loop(
            0, nfull, lambda c, s: _insert_topk(s, load(c)), st0)
        if rem:
            st = _insert_topk(st, load_rem())
        out_ref[:, 0:K] = _tree_topk(st, K)


def _bn(x, g, b):
    mu = jnp.mean(x, axis=0, keepdims=True)
    d = x - mu
    var = jnp.mean(d * d, axis=0, keepdims=True)
    return g * d * jax.lax.rsqrt(var + 1e-5) + b


def _phase2_body(stats_ref, bn1_g_ref, bn1_b_ref, w1_ref, b1_ref,
                 bn2_g_ref, bn2_b_ref, w2_ref, b2_ref,
                 bn3_g_ref, bn3_b_ref, w3_ref, b3_ref, out_ref):
    stats = stats_ref[...]
    topk_l = stats[:, 0:K]
    lse = stats[:, K:K + 1]
    entropy = stats[:, K + 1:K + 2]
    p = jnp.exp(topk_l - lse)

    feats = [entropy]
    for i in range(K):
        feats.append(p[:, i:i + 1] - p)
    x = jnp.concatenate(feats, axis=1)

    h = _bn(x, bn1_g_ref[...], bn1_b_ref[...])
    h = jax.lax.dot_general(h, w1_ref[...], (((1,), (1,)), ((), ())),
                            preferred_element_type=jnp.float32) + b1_ref[...]
    h = _bn(h, bn2_g_ref[...], bn2_b_ref[...])
    h = jnp.maximum(h, 0.0)
    h = jax.lax.dot_general(h, w2_ref[...], (((1,), (1,)), ((), ())),
                            preferred_element_type=jnp.float32) + b2_ref[...]
    h = _bn(h, bn3_g_ref[...], bn3_b_ref[...])
    out_ref[...] = jax.lax.dot_general(
        h, w3_ref[...], (((1,), (1,)), ((), ())),
        preferred_element_type=jnp.float32) + b3_ref[...]


@jax.jit
def kernel(logits, ft, bn1_g, bn1_b, W1, b1, bn2_g, bn2_b, W2, b2,
           bn3_g, bn3_b, W3, b3):
    del ft
    b, v = logits.shape

    stats = pl.pallas_call(
        _phase1_body,
        grid=(b // BLK_B,),
        in_specs=[pl.BlockSpec((BLK_B, v), lambda i: (i, 0))],
        out_specs=pl.BlockSpec((BLK_B, K + 2), lambda i: (i, 0)),
        out_shape=jax.ShapeDtypeStruct((b, K + 2), jnp.float32),
        compiler_params=pltpu.CompilerParams(
            dimension_semantics=("parallel",)),
    )(logits)

    row = lambda a: a.reshape(1, -1)
    gate = pl.pallas_call(
        _phase2_body,
        out_shape=jax.ShapeDtypeStruct((b, 2), jnp.float32),
    )(stats, row(bn1_g), row(bn1_b), W1, row(b1),
      row(bn2_g), row(bn2_b), W2, row(b2),
      row(bn3_g), row(bn3_b), W3, row(b3))
    return gate


# speedup vs baseline: 1.1040x; 1.1040x over previous
import jax
import jax.numpy as jnp
from jax.experimental import pallas as pl
from jax.experimental.pallas import tpu as pltpu

NEG = -3.0e38
K = 20
BLK_B = 8
W = 256
S_FAST = 5


def _insert_topk(state, x):
    new_state = []
    cur = x
    for s in state:
        hi = jnp.maximum(s, cur)
        cur = jnp.minimum(s, cur)
        new_state.append(hi)
    return new_state


def _extract_topk(cand, k):
    r, c = cand.shape
    lane = jax.lax.broadcasted_iota(jnp.int32, (r, c), 1)
    big = jnp.int32(2**30)
    work = cand
    outs = []
    for _ in range(k):
        g = jnp.max(work, axis=1, keepdims=True)
        outs.append(g)
        eq = work == g
        idx = jnp.min(jnp.where(eq, lane, big), axis=1, keepdims=True)
        work = jnp.where(lane == idx, NEG, work)
    return jnp.concatenate(outs, axis=1)


def _merge_sorted(a, b, out_len):
    la, lb = len(a), len(b)
    out = []
    for i in range(out_len):
        terms = []
        if i < la:
            terms.append(a[i])
        if i < lb:
            terms.append(b[i])
        for j in range(i):
            kk = i - 1 - j
            if j < la and kk < lb:
                terms.append(jnp.minimum(a[j], b[kk]))
        while len(terms) > 1:
            terms = [jnp.maximum(terms[t], terms[t + 1])
                     for t in range(0, len(terms) - 1, 2)] + (
                         [terms[-1]] if len(terms) % 2 else [])
        out.append(terms[0])
    return out


def _tree_topk(state, k):
    w = state[0].shape[1]
    while w > 1:
        half = w // 2
        a = [s[:, :half] for s in state]
        b = [s[:, half:] for s in state]
        state = _merge_sorted(a, b, min(2 * len(state), k))
        w = half
    return jnp.concatenate(state, axis=1)


def _phase1_body(x_ref, out_ref):
    v = x_ref.shape[1]
    nfull = v // W
    rem = v % W

    def load(c):
        return x_ref[:, pl.ds(c * W, W)]

    def load_rem():
        xr = x_ref[:, pl.ds(nfull * W, rem)]
        pad = jnp.full((BLK_B, W - rem), NEG, jnp.float32)
        return jnp.concatenate([xr, pad], axis=1)

    state0 = [jnp.full((BLK_B, W), NEG, jnp.float32) for _ in range(S_FAST)]
    ngroup = nfull // 4

    def sort4(a, b, c, d):
        p = [jnp.maximum(a, b), jnp.minimum(a, b)]
        q = [jnp.maximum(c, d), jnp.minimum(c, d)]
        return _merge_sorted(p, q, 4)

    def acc_one(x, carry):
        s_acc, t_acc = carry
        e = jnp.exp(x)
        return s_acc + e, t_acc + e * x

    zero = jnp.zeros((BLK_B, W), jnp.float32)

    def body1(t, carry):
        state, accs = carry
        xs = [load(4 * t + u) for u in range(4)]
        g = sort4(*xs)
        state = _merge_sorted(state, g, S_FAST)
        accs = tuple(acc_one(xs[u], accs[u]) for u in range(4))
        return state, accs

    state, accs = jax.lax.fori_loop(0, ngroup, body1,
                                    (state0, ((zero, zero),) * 4))
    accs = list(accs)
    tail = [load(c) for c in range(4 * ngroup, nfull)]
    if rem:
        tail.append(load_rem())
    for u, x in enumerate(tail):
        state = _merge_sorted(state, [x], S_FAST)
        accs[u] = acc_one(x, accs[u])

    m = jnp.max(state[0], axis=1, keepdims=True)

    topk = _tree_topk(state, K)
    tau = topk[:, K - 1:K]
    bad = jnp.any(state[S_FAST - 1] >= tau)
    bad = jnp.logical_or(bad, jnp.any(jnp.abs(m) > 60.0))

    s_tot = (accs[0][0] + accs[1][0]) + (accs[2][0] + accs[3][0])
    t_tot = (accs[0][1] + accs[1][1]) + (accs[2][1] + accs[3][1])
    s = jnp.sum(s_tot, axis=1, keepdims=True)
    t = jnp.sum(t_tot, axis=1, keepdims=True)
    lse = jnp.log(s)
    entropy = lse - t / s

    out_ref[...] = jnp.concatenate([topk, lse, entropy], axis=1)

    @pl.when(bad)
    def _exact_fallback():
        st0 = [jnp.full((BLK_B, W), NEG, jnp.float32) for _ in range(K)]
        st = jax.lax.fori_loop(
            0, nfull, lambda c, s: _insert_topk(s, load(c)), st0)
        if rem:
            st = _insert_topk(st, load_rem())
        topk_x = _tree_topk(st, K)
        mx = topk_x[:, 0:1]

        def body2(c, carry):
            s_acc, t_acc = carry
            x = load(c)
            e = jnp.exp(x - mx)
            return s_acc + e, t_acc + e * x

        s_acc, t_acc = jax.lax.fori_loop(0, nfull, body2, (zero, zero))
        if rem:
            xr = load_rem()
            e = jnp.exp(xr - mx)
            s_acc, t_acc = s_acc + e, t_acc + e * xr
        sx = jnp.sum(s_acc, axis=1, keepdims=True)
        tx = jnp.sum(t_acc, axis=1, keepdims=True)
        lse_x = mx + jnp.log(sx)
        ent_x = lse_x - (tx / sx)
        out_ref[...] = jnp.concatenate([topk_x, lse_x, ent_x], axis=1)


def _bn(x, g, b):
    mu = jnp.mean(x, axis=0, keepdims=True)
    d = x - mu
    var = jnp.mean(d * d, axis=0, keepdims=True)
    return g * d * jax.lax.rsqrt(var + 1e-5) + b


def _phase2_body(stats_ref, bn1_g_ref, bn1_b_ref, w1_ref, b1_ref,
                 bn2_g_ref, bn2_b_ref, w2_ref, b2_ref,
                 bn3_g_ref, bn3_b_ref, w3_ref, b3_ref, out_ref):
    stats = stats_ref[...]
    topk_l = stats[:, 0:K]
    lse = stats[:, K:K + 1]
    entropy = stats[:, K + 1:K + 2]
    p = jnp.exp(topk_l - lse)

    feats = [entropy]
    for i in range(K):
        feats.append(p[:, i:i + 1] - p)
    x = jnp.concatenate(feats, axis=1)

    h = _bn(x, bn1_g_ref[...], bn1_b_ref[...])
    h = jax.lax.dot_general(h, w1_ref[...], (((1,), (1,)), ((), ())),
                            preferred_element_type=jnp.float32) + b1_ref[...]
    h = _bn(h, bn2_g_ref[...], bn2_b_ref[...])
    h = jnp.maximum(h, 0.0)
    h = jax.lax.dot_general(h, w2_ref[...], (((1,), (1,)), ((), ())),
                            preferred_element_type=jnp.float32) + b2_ref[...]
    h = _bn(h, bn3_g_ref[...], bn3_b_ref[...])
    out_ref[...] = jax.lax.dot_general(
        h, w3_ref[...], (((1,), (1,)), ((), ())),
        preferred_element_type=jnp.float32) + b3_ref[...]


@jax.jit
def kernel(logits, ft, bn1_g, bn1_b, W1, b1, bn2_g, bn2_b, W2, b2,
           bn3_g, bn3_b, W3, b3):
    del ft
    b, v = logits.shape

    stats = pl.pallas_call(
        _phase1_body,
        grid=(b // BLK_B,),
        in_specs=[pl.BlockSpec((BLK_B, v), lambda i: (i, 0))],
        out_specs=pl.BlockSpec((BLK_B, K + 2), lambda i: (i, 0)),
        out_shape=jax.ShapeDtypeStruct((b, K + 2), jnp.float32),
        compiler_params=pltpu.CompilerParams(
            dimension_semantics=("parallel",)),
    )(logits)

    row = lambda a: a.reshape(1, -1)
    gate = pl.pallas_call(
        _phase2_body,
        out_shape=jax.ShapeDtypeStruct((b, 2), jnp.float32),
    )(stats, row(bn1_g), row(bn1_b), W1, row(b1),
      row(bn2_g), row(bn2_b), W2, row(b2),
      row(bn3_g), row(bn3_b), W3, row(b3))
    return gate


# speedup vs baseline: 1.1233x; 1.0175x over previous
import jax
import jax.numpy as jnp
from jax.experimental import pallas as pl
from jax.experimental.pallas import tpu as pltpu

NEG = -3.0e38
K = 20
BLK_B = 8
W = 256
S_FAST = 5


def _insert_topk(state, x):
    new_state = []
    cur = x
    for s in state:
        hi = jnp.maximum(s, cur)
        cur = jnp.minimum(s, cur)
        new_state.append(hi)
    return new_state


def _extract_topk(cand, k):
    r, c = cand.shape
    lane = jax.lax.broadcasted_iota(jnp.int32, (r, c), 1)
    big = jnp.int32(2**30)
    work = cand
    outs = []
    for _ in range(k):
        g = jnp.max(work, axis=1, keepdims=True)
        outs.append(g)
        eq = work == g
        idx = jnp.min(jnp.where(eq, lane, big), axis=1, keepdims=True)
        work = jnp.where(lane == idx, NEG, work)
    return jnp.concatenate(outs, axis=1)


def _merge_sorted(a, b, out_len):
    la, lb = len(a), len(b)
    out = []
    for i in range(out_len):
        terms = []
        if i < la:
            terms.append(a[i])
        if i < lb:
            terms.append(b[i])
        for j in range(i):
            kk = i - 1 - j
            if j < la and kk < lb:
                terms.append(jnp.minimum(a[j], b[kk]))
        while len(terms) > 1:
            terms = [jnp.maximum(terms[t], terms[t + 1])
                     for t in range(0, len(terms) - 1, 2)] + (
                         [terms[-1]] if len(terms) % 2 else [])
        out.append(terms[0])
    return out


def _tree_topk(state, k):
    w = state[0].shape[1]
    while w > 1:
        half = w // 2
        a = [s[:, :half] for s in state]
        b = [s[:, half:] for s in state]
        state = _merge_sorted(a, b, min(2 * len(state), k))
        w = half
    return jnp.concatenate(state, axis=1)


def _phase1_body(x_ref, out_ref):
    v = x_ref.shape[1]
    nfull = v // W
    rem = v % W

    def load(c):
        return x_ref[:, pl.ds(c * W, W)]

    def load_rem():
        xr = x_ref[:, pl.ds(nfull * W, rem)]
        pad = jnp.full((BLK_B, W - rem), NEG, jnp.float32)
        return jnp.concatenate([xr, pad], axis=1)

    state0 = [jnp.full((BLK_B, W), NEG, jnp.float32) for _ in range(S_FAST)]
    ngroup = nfull // 4

    def sort4(a, b, c, d):
        p = [jnp.maximum(a, b), jnp.minimum(a, b)]
        q = [jnp.maximum(c, d), jnp.minimum(c, d)]
        return _merge_sorted(p, q, 4)

    zero = jnp.zeros((BLK_B, W), jnp.float32)

    def body1(t, carry):
        state, s_acc, t_acc = carry
        xs = [load(4 * t + u) for u in range(4)]
        g = sort4(*xs)
        state = _merge_sorted(state, g, S_FAST)
        es = [jnp.exp(x) for x in xs]
        s_acc = s_acc + ((es[0] + es[1]) + (es[2] + es[3]))
        t_acc = t_acc + ((es[0] * xs[0] + es[1] * xs[1])
                         + (es[2] * xs[2] + es[3] * xs[3]))
        return state, s_acc, t_acc

    state, s_acc, t_acc = jax.lax.fori_loop(0, ngroup, body1,
                                            (state0, zero, zero))
    tail = [load(c) for c in range(4 * ngroup, nfull)]
    if rem:
        tail.append(load_rem())
    for x in tail:
        state = _merge_sorted(state, [x], S_FAST)
        e = jnp.exp(x)
        s_acc = s_acc + e
        t_acc = t_acc + e * x

    m = jnp.max(state[0], axis=1, keepdims=True)

    topk = _tree_topk(state, K)
    tau = topk[:, K - 1:K]
    bad = jnp.any(state[S_FAST - 1] >= tau)
    bad = jnp.logical_or(bad, jnp.any(jnp.abs(m) > 60.0))

    s = jnp.sum(s_acc, axis=1, keepdims=True)
    t = jnp.sum(t_acc, axis=1, keepdims=True)
    lse = jnp.log(s)
    entropy = lse - t / s

    out_ref[...] = jnp.concatenate([topk, lse, entropy], axis=1)

    @pl.when(bad)
    def _exact_fallback():
        st0 = [jnp.full((BLK_B, W), NEG, jnp.float32) for _ in range(K)]
        st = jax.lax.fori_loop(
            0, nfull, lambda c, s: _insert_topk(s, load(c)), st0)
        if rem:
            st = _insert_topk(st, load_rem())
        topk_x = _tree_topk(st, K)
        mx = topk_x[:, 0:1]

        def body2(c, carry):
            s_acc, t_acc = carry
            x = load(c)
            e = jnp.exp(x - mx)
            return s_acc + e, t_acc + e * x

        s_acc, t_acc = jax.lax.fori_loop(0, nfull, body2, (zero, zero))
        if rem:
            xr = load_rem()
            e = jnp.exp(xr - mx)
            s_acc, t_acc = s_acc + e, t_acc + e * xr
        sx = jnp.sum(s_acc, axis=1, keepdims=True)
        tx = jnp.sum(t_acc, axis=1, keepdims=True)
        lse_x = mx + jnp.log(sx)
        ent_x = lse_x - (tx / sx)
        out_ref[...] = jnp.concatenate([topk_x, lse_x, ent_x], axis=1)


def _bn(x, g, b):
    mu = jnp.mean(x, axis=0, keepdims=True)
    d = x - mu
    var = jnp.mean(d * d, axis=0, keepdims=True)
    return g * d * jax.lax.rsqrt(var + 1e-5) + b


def _phase2_body(stats_ref, bn1_g_ref, bn1_b_ref, w1_ref, b1_ref,
                 bn2_g_ref, bn2_b_ref, w2_ref, b2_ref,
                 bn3_g_ref, bn3_b_ref, w3_ref, b3_ref, out_ref):
    stats = stats_ref[...]
    topk_l = stats[:, 0:K]
    lse = stats[:, K:K + 1]
    entropy = stats[:, K + 1:K + 2]
    p = jnp.exp(topk_l - lse)

    feats = [entropy]
    for i in range(K):
        feats.append(p[:, i:i + 1] - p)
    x = jnp.concatenate(feats, axis=1)

    h = _bn(x, bn1_g_ref[...], bn1_b_ref[...])
    h = jax.lax.dot_general(h, w1_ref[...], (((1,), (1,)), ((), ())),
                            preferred_element_type=jnp.float32) + b1_ref[...]
    h = _bn(h, bn2_g_ref[...], bn2_b_ref[...])
    h = jnp.maximum(h, 0.0)
    h = jax.lax.dot_general(h, w2_ref[...], (((1,), (1,)), ((), ())),
                            preferred_element_type=jnp.float32) + b2_ref[...]
    h = _bn(h, bn3_g_ref[...], bn3_b_ref[...])
    out_ref[...] = jax.lax.dot_general(
        h, w3_ref[...], (((1,), (1,)), ((), ())),
        preferred_element_type=jnp.float32) + b3_ref[...]


@jax.jit
def kernel(logits, ft, bn1_g, bn1_b, W1, b1, bn2_g, bn2_b, W2, b2,
           bn3_g, bn3_b, W3, b3):
    del ft
    b, v = logits.shape

    stats = pl.pallas_call(
        _phase1_body,
        grid=(b // BLK_B,),
        in_specs=[pl.BlockSpec((BLK_B, v), lambda i: (i, 0))],
        out_specs=pl.BlockSpec((BLK_B, K + 2), lambda i: (i, 0)),
        out_shape=jax.ShapeDtypeStruct((b, K + 2), jnp.float32),
        compiler_params=pltpu.CompilerParams(
            dimension_semantics=("parallel",)),
    )(logits)

    row = lambda a: a.reshape(1, -1)
    gate = pl.pallas_call(
        _phase2_body,
        out_shape=jax.ShapeDtypeStruct((b, 2), jnp.float32),
    )(stats, row(bn1_g), row(bn1_b), W1, row(b1),
      row(bn2_g), row(bn2_b), W2, row(b2),
      row(bn3_g), row(bn3_b), W3, row(b3))
    return gate


# speedup vs baseline: 1.2451x; 1.1084x over previous
import jax
import jax.numpy as jnp
from jax.experimental import pallas as pl
from jax.experimental.pallas import tpu as pltpu

NEG = -3.0e38
K = 20
BLK_B = 8
W = 256
S_FAST = 5


def _insert_topk(state, x):
    new_state = []
    cur = x
    for s in state:
        hi = jnp.maximum(s, cur)
        cur = jnp.minimum(s, cur)
        new_state.append(hi)
    return new_state


def _extract_topk(cand, k):
    r, c = cand.shape
    lane = jax.lax.broadcasted_iota(jnp.int32, (r, c), 1)
    big = jnp.int32(2**30)
    work = cand
    outs = []
    for _ in range(k):
        g = jnp.max(work, axis=1, keepdims=True)
        outs.append(g)
        eq = work == g
        idx = jnp.min(jnp.where(eq, lane, big), axis=1, keepdims=True)
        work = jnp.where(lane == idx, NEG, work)
    return jnp.concatenate(outs, axis=1)


def _merge_sorted(a, b, out_len):
    la, lb = len(a), len(b)
    out = []
    for i in range(out_len):
        terms = []
        if i < la:
            terms.append(a[i])
        if i < lb:
            terms.append(b[i])
        for j in range(i):
            kk = i - 1 - j
            if j < la and kk < lb:
                terms.append(jnp.minimum(a[j], b[kk]))
        while len(terms) > 1:
            terms = [jnp.maximum(terms[t], terms[t + 1])
                     for t in range(0, len(terms) - 1, 2)] + (
                         [terms[-1]] if len(terms) % 2 else [])
        out.append(terms[0])
    return out


def _tree_topk(state, k):
    w = state[0].shape[1]
    while w > 1:
        half = w // 2
        a = [s[:, :half] for s in state]
        b = [s[:, half:] for s in state]
        state = _merge_sorted(a, b, min(2 * len(state), k))
        w = half
    return jnp.concatenate(state, axis=1)


def _phase1_body(x_ref, out_ref):
    v = x_ref.shape[1]
    nfull = v // W
    rem = v % W

    def load(c):
        return x_ref[:, pl.ds(c * W, W)]

    def load_rem():
        xr = x_ref[:, pl.ds(nfull * W, rem)]
        pad = jnp.full((BLK_B, W - rem), NEG, jnp.float32)
        return jnp.concatenate([xr, pad], axis=1)

    state0 = [jnp.full((BLK_B, W), NEG, jnp.float32) for _ in range(S_FAST)]
    ngroup = nfull // 4

    def sort4(a, b, c, d):
        p = [jnp.maximum(a, b), jnp.minimum(a, b)]
        q = [jnp.maximum(c, d), jnp.minimum(c, d)]
        return _merge_sorted(p, q, 4)

    zero = jnp.zeros((BLK_B, W), jnp.float32)

    def group_step(state, s_acc, t_acc, xs):
        g = sort4(*xs)
        state = _merge_sorted(state, g, S_FAST)
        es = [jnp.exp(x) for x in xs]
        s_acc = s_acc + ((es[0] + es[1]) + (es[2] + es[3]))
        t_acc = t_acc + ((es[0] * xs[0] + es[1] * xs[1])
                         + (es[2] * xs[2] + es[3] * xs[3]))
        return state, s_acc, t_acc

    def body1(t, carry):
        state, s_acc, t_acc = carry
        for half in range(2):
            xs = [load(8 * t + 4 * half + u) for u in range(4)]
            state, s_acc, t_acc = group_step(state, s_acc, t_acc, xs)
        return state, s_acc, t_acc

    state, s_acc, t_acc = jax.lax.fori_loop(0, ngroup // 2, body1,
                                            (state0, zero, zero))
    for t8 in range(ngroup // 2 * 2, ngroup):
        xs = [load(4 * t8 + u) for u in range(4)]
        state, s_acc, t_acc = group_step(state, s_acc, t_acc, xs)
    tail = [load(c) for c in range(4 * ngroup, nfull)]
    if rem:
        tail.append(load_rem())
    for x in tail:
        state = _merge_sorted(state, [x], S_FAST)
        e = jnp.exp(x)
        s_acc = s_acc + e
        t_acc = t_acc + e * x

    m = jnp.max(state[0], axis=1, keepdims=True)

    topk = _tree_topk(state, K)
    tau = topk[:, K - 1:K]
    bad = jnp.any(state[S_FAST - 1] >= tau)
    bad = jnp.logical_or(bad, jnp.any(jnp.abs(m) > 60.0))

    s = jnp.sum(s_acc, axis=1, keepdims=True)
    t = jnp.sum(t_acc, axis=1, keepdims=True)
    lse = jnp.log(s)
    entropy = lse - t / s

    out_ref[...] = jnp.concatenate([topk, lse, entropy], axis=1)

    @pl.when(bad)
    def _exact_fallback():
        st0 = [jnp.full((BLK_B, W), NEG, jnp.float32) for _ in range(K)]
        st = jax.lax.fori_loop(
            0, nfull, lambda c, s: _insert_topk(s, load(c)), st0)
        if rem:
            st = _insert_topk(st, load_rem())
        topk_x = _tree_topk(st, K)
        mx = topk_x[:, 0:1]

        def body2(c, carry):
            s_acc, t_acc = carry
            x = load(c)
            e = jnp.exp(x - mx)
            return s_acc + e, t_acc + e * x

        s_acc, t_acc = jax.lax.fori_loop(0, nfull, body2, (zero, zero))
        if rem:
            xr = load_rem()
            e = jnp.exp(xr - mx)
            s_acc, t_acc = s_acc + e, t_acc + e * xr
        sx = jnp.sum(s_acc, axis=1, keepdims=True)
        tx = jnp.sum(t_acc, axis=1, keepdims=True)
        lse_x = mx + jnp.log(sx)
        ent_x = lse_x - (tx / sx)
        out_ref[...] = jnp.concatenate([topk_x, lse_x, ent_x], axis=1)


def _bn(x, g, b):
    mu = jnp.mean(x, axis=0, keepdims=True)
    d = x - mu
    var = jnp.mean(d * d, axis=0, keepdims=True)
    return g * d * jax.lax.rsqrt(var + 1e-5) + b


def _phase2_body(stats_ref, bn1_g_ref, bn1_b_ref, w1_ref, b1_ref,
                 bn2_g_ref, bn2_b_ref, w2_ref, b2_ref,
                 bn3_g_ref, bn3_b_ref, w3_ref, b3_ref, out_ref):
    stats = stats_ref[...]
    topk_l = stats[:, 0:K]
    lse = stats[:, K:K + 1]
    entropy = stats[:, K + 1:K + 2]
    p = jnp.exp(topk_l - lse)

    feats = [entropy]
    for i in range(K):
        feats.append(p[:, i:i + 1] - p)
    x = jnp.concatenate(feats, axis=1)

    h = _bn(x, bn1_g_ref[...], bn1_b_ref[...])
    h = jax.lax.dot_general(h, w1_ref[...], (((1,), (1,)), ((), ())),
                            preferred_element_type=jnp.float32) + b1_ref[...]
    h = _bn(h, bn2_g_ref[...], bn2_b_ref[...])
    h = jnp.maximum(h, 0.0)
    h = jax.lax.dot_general(h, w2_ref[...], (((1,), (1,)), ((), ())),
                            preferred_element_type=jnp.float32) + b2_ref[...]
    h = _bn(h, bn3_g_ref[...], bn3_b_ref[...])
    out_ref[...] = jax.lax.dot_general(
        h, w3_ref[...], (((1,), (1,)), ((), ())),
        preferred_element_type=jnp.float32) + b3_ref[...]


@jax.jit
def kernel(logits, ft, bn1_g, bn1_b, W1, b1, bn2_g, bn2_b, W2, b2,
           bn3_g, bn3_b, W3, b3):
    del ft
    b, v = logits.shape

    stats = pl.pallas_call(
        _phase1_body,
        grid=(b // BLK_B,),
        in_specs=[pl.BlockSpec((BLK_B, v), lambda i: (i, 0))],
        out_specs=pl.BlockSpec((BLK_B, K + 2), lambda i: (i, 0)),
        out_shape=jax.ShapeDtypeStruct((b, K + 2), jnp.float32),
        compiler_params=pltpu.CompilerParams(
            dimension_semantics=("parallel",)),
    )(logits)

    row = lambda a: a.reshape(1, -1)
    gate = pl.pallas_call(
        _phase2_body,
        out_shape=jax.ShapeDtypeStruct((b, 2), jnp.float32),
    )(stats, row(bn1_g), row(bn1_b), W1, row(b1),
      row(bn2_g), row(bn2_b), W2, row(b2),
      row(bn3_g), row(bn3_b), W3, row(b3))
    return gate


# speedup vs baseline: 1.3114x; 1.0533x over previous
import jax
import jax.numpy as jnp
from jax.experimental import pallas as pl
from jax.experimental.pallas import tpu as pltpu

NEG = -3.0e38
K = 20
BLK_B = 8
W = 256
S_FAST = 5


def _insert_topk(state, x):
    new_state = []
    cur = x
    for s in state:
        hi = jnp.maximum(s, cur)
        cur = jnp.minimum(s, cur)
        new_state.append(hi)
    return new_state


def _extract_topk(cand, k):
    r, c = cand.shape
    lane = jax.lax.broadcasted_iota(jnp.int32, (r, c), 1)
    big = jnp.int32(2**30)
    work = cand
    outs = []
    for _ in range(k):
        g = jnp.max(work, axis=1, keepdims=True)
        outs.append(g)
        eq = work == g
        idx = jnp.min(jnp.where(eq, lane, big), axis=1, keepdims=True)
        work = jnp.where(lane == idx, NEG, work)
    return jnp.concatenate(outs, axis=1)


def _merge_sorted(a, b, out_len):
    la, lb = len(a), len(b)
    out = []
    for i in range(out_len):
        terms = []
        if i < la:
            terms.append(a[i])
        if i < lb:
            terms.append(b[i])
        for j in range(i):
            kk = i - 1 - j
            if j < la and kk < lb:
                terms.append(jnp.minimum(a[j], b[kk]))
        while len(terms) > 1:
            terms = [jnp.maximum(terms[t], terms[t + 1])
                     for t in range(0, len(terms) - 1, 2)] + (
                         [terms[-1]] if len(terms) % 2 else [])
        out.append(terms[0])
    return out


def _tree_topk(state, k):
    w = state[0].shape[1]
    while w > 1:
        half = w // 2
        a = [s[:, :half] for s in state]
        b = [s[:, half:] for s in state]
        state = _merge_sorted(a, b, min(2 * len(state), k))
        w = half
    return jnp.concatenate(state, axis=1)


def _phase1_body(x_ref, out_ref):
    v = x_ref.shape[1]
    nfull = v // W
    rem = v % W

    def load(c):
        return x_ref[:, pl.ds(c * W, W)]

    def load_rem():
        xr = x_ref[:, pl.ds(nfull * W, rem)]
        pad = jnp.full((BLK_B, W - rem), NEG, jnp.float32)
        return jnp.concatenate([xr, pad], axis=1)

    state0 = [jnp.full((BLK_B, W), NEG, jnp.float32) for _ in range(S_FAST)]
    ngroup = nfull // 4

    def sort4(a, b, c, d):
        p = [jnp.maximum(a, b), jnp.minimum(a, b)]
        q = [jnp.maximum(c, d), jnp.minimum(c, d)]
        return _merge_sorted(p, q, 4)

    zero = jnp.zeros((BLK_B, W), jnp.float32)

    def group_step(state, s_acc, t_acc, xs):
        g = sort4(*xs)
        state = _merge_sorted(state, g, S_FAST)
        es = [jnp.exp(x) for x in xs]
        s_acc = s_acc + ((es[0] + es[1]) + (es[2] + es[3]))
        t_acc = t_acc + ((es[0] * xs[0] + es[1] * xs[1])
                         + (es[2] * xs[2] + es[3] * xs[3]))
        return state, s_acc, t_acc

    UNROLL = 4

    def body1(t, carry):
        state, s_acc, t_acc = carry
        for half in range(UNROLL):
            xs = [load(4 * (UNROLL * t + half) + u) for u in range(4)]
            state, s_acc, t_acc = group_step(state, s_acc, t_acc, xs)
        return state, s_acc, t_acc

    state, s_acc, t_acc = jax.lax.fori_loop(0, ngroup // UNROLL, body1,
                                            (state0, zero, zero))
    for t8 in range(ngroup // UNROLL * UNROLL, ngroup):
        xs = [load(4 * t8 + u) for u in range(4)]
        state, s_acc, t_acc = group_step(state, s_acc, t_acc, xs)
    tail = [load(c) for c in range(4 * ngroup, nfull)]
    if rem:
        tail.append(load_rem())
    for x in tail:
        state = _merge_sorted(state, [x], S_FAST)
        e = jnp.exp(x)
        s_acc = s_acc + e
        t_acc = t_acc + e * x

    m = jnp.max(state[0], axis=1, keepdims=True)

    topk = _tree_topk(state, K)
    tau = topk[:, K - 1:K]
    bad = jnp.any(state[S_FAST - 1] >= tau)
    bad = jnp.logical_or(bad, jnp.any(jnp.abs(m) > 60.0))

    s = jnp.sum(s_acc, axis=1, keepdims=True)
    t = jnp.sum(t_acc, axis=1, keepdims=True)
    lse = jnp.log(s)
    entropy = lse - t / s

    out_ref[...] = jnp.concatenate([topk, lse, entropy], axis=1)

    @pl.when(bad)
    def _exact_fallback():
        st0 = [jnp.full((BLK_B, W), NEG, jnp.float32) for _ in range(K)]
        st = jax.lax.fori_loop(
            0, nfull, lambda c, s: _insert_topk(s, load(c)), st0)
        if rem:
            st = _insert_topk(st, load_rem())
        topk_x = _tree_topk(st, K)
        mx = topk_x[:, 0:1]

        def body2(c, carry):
            s_acc, t_acc = carry
            x = load(c)
            e = jnp.exp(x - mx)
            return s_acc + e, t_acc + e * x

        s_acc, t_acc = jax.lax.fori_loop(0, nfull, body2, (zero, zero))
        if rem:
            xr = load_rem()
            e = jnp.exp(xr - mx)
            s_acc, t_acc = s_acc + e, t_acc + e * xr
        sx = jnp.sum(s_acc, axis=1, keepdims=True)
        tx = jnp.sum(t_acc, axis=1, keepdims=True)
        lse_x = mx + jnp.log(sx)
        ent_x = lse_x - (tx / sx)
        out_ref[...] = jnp.concatenate([topk_x, lse_x, ent_x], axis=1)


def _bn(x, g, b):
    mu = jnp.mean(x, axis=0, keepdims=True)
    d = x - mu
    var = jnp.mean(d * d, axis=0, keepdims=True)
    return g * d * jax.lax.rsqrt(var + 1e-5) + b


def _phase2_body(stats_ref, bn1_g_ref, bn1_b_ref, w1_ref, b1_ref,
                 bn2_g_ref, bn2_b_ref, w2_ref, b2_ref,
                 bn3_g_ref, bn3_b_ref, w3_ref, b3_ref, out_ref):
    stats = stats_ref[...]
    topk_l = stats[:, 0:K]
    lse = stats[:, K:K + 1]
    entropy = stats[:, K + 1:K + 2]
    p = jnp.exp(topk_l - lse)

    feats = [entropy]
    for i in range(K):
        feats.append(p[:, i:i + 1] - p)
    x = jnp.concatenate(feats, axis=1)

    h = _bn(x, bn1_g_ref[...], bn1_b_ref[...])
    h = jax.lax.dot_general(h, w1_ref[...], (((1,), (1,)), ((), ())),
                            preferred_element_type=jnp.float32) + b1_ref[...]
    h = _bn(h, bn2_g_ref[...], bn2_b_ref[...])
    h = jnp.maximum(h, 0.0)
    h = jax.lax.dot_general(h, w2_ref[...], (((1,), (1,)), ((), ())),
                            preferred_element_type=jnp.float32) + b2_ref[...]
    h = _bn(h, bn3_g_ref[...], bn3_b_ref[...])
    out_ref[...] = jax.lax.dot_general(
        h, w3_ref[...], (((1,), (1,)), ((), ())),
        preferred_element_type=jnp.float32) + b3_ref[...]


@jax.jit
def kernel(logits, ft, bn1_g, bn1_b, W1, b1, bn2_g, bn2_b, W2, b2,
           bn3_g, bn3_b, W3, b3):
    del ft
    b, v = logits.shape

    stats = pl.pallas_call(
        _phase1_body,
        grid=(b // BLK_B,),
        in_specs=[pl.BlockSpec((BLK_B, v), lambda i: (i, 0))],
        out_specs=pl.BlockSpec((BLK_B, K + 2), lambda i: (i, 0)),
        out_shape=jax.ShapeDtypeStruct((b, K + 2), jnp.float32),
        compiler_params=pltpu.CompilerParams(
            dimension_semantics=("parallel",)),
    )(logits)

    row = lambda a: a.reshape(1, -1)
    gate = pl.pallas_call(
        _phase2_body,
        out_shape=jax.ShapeDtypeStruct((b, 2), jnp.float32),
    )(stats, row(bn1_g), row(bn1_b), W1, row(b1),
      row(bn2_g), row(bn2_b), W2, row(b2),
      row(bn3_g), row(bn3_b), W3, row(b3))
    return gate


# speedup vs baseline: 1.3378x; 1.0201x over previous
import jax
import jax.numpy as jnp
from jax.experimental import pallas as pl
from jax.experimental.pallas import tpu as pltpu

NEG = -3.0e38
K = 20
BLK_B = 8
W = 256
S_FAST = 5


def _insert_topk(state, x):
    new_state = []
    cur = x
    for s in state:
        hi = jnp.maximum(s, cur)
        cur = jnp.minimum(s, cur)
        new_state.append(hi)
    return new_state


def _extract_topk(cand, k):
    r, c = cand.shape
    lane = jax.lax.broadcasted_iota(jnp.int32, (r, c), 1)
    big = jnp.int32(2**30)
    work = cand
    outs = []
    for _ in range(k):
        g = jnp.max(work, axis=1, keepdims=True)
        outs.append(g)
        eq = work == g
        idx = jnp.min(jnp.where(eq, lane, big), axis=1, keepdims=True)
        work = jnp.where(lane == idx, NEG, work)
    return jnp.concatenate(outs, axis=1)


def _merge_sorted(a, b, out_len):
    la, lb = len(a), len(b)
    out = []
    for i in range(out_len):
        terms = []
        if i < la:
            terms.append(a[i])
        if i < lb:
            terms.append(b[i])
        for j in range(i):
            kk = i - 1 - j
            if j < la and kk < lb:
                terms.append(jnp.minimum(a[j], b[kk]))
        while len(terms) > 1:
            terms = [jnp.maximum(terms[t], terms[t + 1])
                     for t in range(0, len(terms) - 1, 2)] + (
                         [terms[-1]] if len(terms) % 2 else [])
        out.append(terms[0])
    return out


def _tree_topk(state, k):
    w = state[0].shape[1]
    while w > 1:
        half = w // 2
        a = [s[:, :half] for s in state]
        b = [s[:, half:] for s in state]
        state = _merge_sorted(a, b, min(2 * len(state), k))
        w = half
    return jnp.concatenate(state, axis=1)


def _phase1_body(x_ref, out_ref):
    v = x_ref.shape[1]
    nfull = v // W
    rem = v % W

    def load(c):
        return x_ref[:, pl.ds(c * W, W)]

    def load_rem():
        xr = x_ref[:, pl.ds(nfull * W, rem)]
        pad = jnp.full((BLK_B, W - rem), NEG, jnp.float32)
        return jnp.concatenate([xr, pad], axis=1)

    state0 = [jnp.full((BLK_B, W), NEG, jnp.float32) for _ in range(S_FAST)]
    ngroup = nfull // 4

    def sort4(a, b, c, d):
        p = [jnp.maximum(a, b), jnp.minimum(a, b)]
        q = [jnp.maximum(c, d), jnp.minimum(c, d)]
        return _merge_sorted(p, q, 4)

    zero = jnp.zeros((BLK_B, W), jnp.float32)

    def group_step(state, s_acc, t_acc, xs):
        g = sort4(*xs)
        state = _merge_sorted(state, g, S_FAST)
        es = [jnp.exp(x) for x in xs]
        s_acc = s_acc + ((es[0] + es[1]) + (es[2] + es[3]))
        t_acc = t_acc + ((es[0] * xs[0] + es[1] * xs[1])
                         + (es[2] * xs[2] + es[3] * xs[3]))
        return state, s_acc, t_acc

    UNROLL = 8

    def body1(t, carry):
        state, s_acc, t_acc = carry
        for half in range(UNROLL):
            xs = [load(4 * (UNROLL * t + half) + u) for u in range(4)]
            state, s_acc, t_acc = group_step(state, s_acc, t_acc, xs)
        return state, s_acc, t_acc

    state, s_acc, t_acc = jax.lax.fori_loop(0, ngroup // UNROLL, body1,
                                            (state0, zero, zero))
    for t8 in range(ngroup // UNROLL * UNROLL, ngroup):
        xs = [load(4 * t8 + u) for u in range(4)]
        state, s_acc, t_acc = group_step(state, s_acc, t_acc, xs)
    tail = [load(c) for c in range(4 * ngroup, nfull)]
    if rem:
        tail.append(load_rem())
    for x in tail:
        state = _merge_sorted(state, [x], S_FAST)
        e = jnp.exp(x)
        s_acc = s_acc + e
        t_acc = t_acc + e * x

    m = jnp.max(state[0], axis=1, keepdims=True)

    topk = _tree_topk(state, K)
    tau = topk[:, K - 1:K]
    bad = jnp.any(state[S_FAST - 1] >= tau)
    bad = jnp.logical_or(bad, jnp.any(jnp.abs(m) > 60.0))

    s = jnp.sum(s_acc, axis=1, keepdims=True)
    t = jnp.sum(t_acc, axis=1, keepdims=True)
    lse = jnp.log(s)
    entropy = lse - t / s

    out_ref[...] = jnp.concatenate([topk, lse, entropy], axis=1)

    @pl.when(bad)
    def _exact_fallback():
        st0 = [jnp.full((BLK_B, W), NEG, jnp.float32) for _ in range(K)]
        st = jax.lax.fori_loop(
            0, nfull, lambda c, s: _insert_topk(s, load(c)), st0)
        if rem:
            st = _insert_topk(st, load_rem())
        topk_x = _tree_topk(st, K)
        mx = topk_x[:, 0:1]

        def body2(c, carry):
            s_acc, t_acc = carry
            x = load(c)
            e = jnp.exp(x - mx)
            return s_acc + e, t_acc + e * x

        s_acc, t_acc = jax.lax.fori_loop(0, nfull, body2, (zero, zero))
        if rem:
            xr = load_rem()
            e = jnp.exp(xr - mx)
            s_acc, t_acc = s_acc + e, t_acc + e * xr
        sx = jnp.sum(s_acc, axis=1, keepdims=True)
        tx = jnp.sum(t_acc, axis=1, keepdims=True)
        lse_x = mx + jnp.log(sx)
        ent_x = lse_x - (tx / sx)
        out_ref[...] = jnp.concatenate([topk_x, lse_x, ent_x], axis=1)


def _bn(x, g, b):
    mu = jnp.mean(x, axis=0, keepdims=True)
    d = x - mu
    var = jnp.mean(d * d, axis=0, keepdims=True)
    return g * d * jax.lax.rsqrt(var + 1e-5) + b


def _phase2_body(stats_ref, bn1_g_ref, bn1_b_ref, w1_ref, b1_ref,
                 bn2_g_ref, bn2_b_ref, w2_ref, b2_ref,
                 bn3_g_ref, bn3_b_ref, w3_ref, b3_ref, out_ref):
    stats = stats_ref[...]
    topk_l = stats[:, 0:K]
    lse = stats[:, K:K + 1]
    entropy = stats[:, K + 1:K + 2]
    p = jnp.exp(topk_l - lse)

    feats = [entropy]
    for i in range(K):
        feats.append(p[:, i:i + 1] - p)
    x = jnp.concatenate(feats, axis=1)

    h = _bn(x, bn1_g_ref[...], bn1_b_ref[...])
    h = jax.lax.dot_general(h, w1_ref[...], (((1,), (1,)), ((), ())),
                            preferred_element_type=jnp.float32) + b1_ref[...]
    h = _bn(h, bn2_g_ref[...], bn2_b_ref[...])
    h = jnp.maximum(h, 0.0)
    h = jax.lax.dot_general(h, w2_ref[...], (((1,), (1,)), ((), ())),
                            preferred_element_type=jnp.float32) + b2_ref[...]
    h = _bn(h, bn3_g_ref[...], bn3_b_ref[...])
    out_ref[...] = jax.lax.dot_general(
        h, w3_ref[...], (((1,), (1,)), ((), ())),
        preferred_element_type=jnp.float32) + b3_ref[...]


@jax.jit
def kernel(logits, ft, bn1_g, bn1_b, W1, b1, bn2_g, bn2_b, W2, b2,
           bn3_g, bn3_b, W3, b3):
    del ft
    b, v = logits.shape

    stats = pl.pallas_call(
        _phase1_body,
        grid=(b // BLK_B,),
        in_specs=[pl.BlockSpec((BLK_B, v), lambda i: (i, 0))],
        out_specs=pl.BlockSpec((BLK_B, K + 2), lambda i: (i, 0)),
        out_shape=jax.ShapeDtypeStruct((b, K + 2), jnp.float32),
        compiler_params=pltpu.CompilerParams(
            dimension_semantics=("parallel",)),
    )(logits)

    row = lambda a: a.reshape(1, -1)
    gate = pl.pallas_call(
        _phase2_body,
        out_shape=jax.ShapeDtypeStruct((b, 2), jnp.float32),
    )(stats, row(bn1_g), row(bn1_b), W1, row(b1),
      row(bn2_g), row(bn2_b), W2, row(b2),
      row(bn3_g), row(bn3_b), W3, row(b3))
    return gate


# speedup vs baseline: 1.3519x; 1.0105x over previous
import jax
import jax.numpy as jnp
from jax.experimental import pallas as pl
from jax.experimental.pallas import tpu as pltpu

NEG = -3.0e38
K = 20
BLK_B = 16
W = 128
S_FAST = 6


def _insert_topk(state, x):
    new_state = []
    cur = x
    for s in state:
        hi = jnp.maximum(s, cur)
        cur = jnp.minimum(s, cur)
        new_state.append(hi)
    return new_state


def _extract_topk(cand, k):
    r, c = cand.shape
    lane = jax.lax.broadcasted_iota(jnp.int32, (r, c), 1)
    big = jnp.int32(2**30)
    work = cand
    outs = []
    for _ in range(k):
        g = jnp.max(work, axis=1, keepdims=True)
        outs.append(g)
        eq = work == g
        idx = jnp.min(jnp.where(eq, lane, big), axis=1, keepdims=True)
        work = jnp.where(lane == idx, NEG, work)
    return jnp.concatenate(outs, axis=1)


def _merge_sorted(a, b, out_len):
    la, lb = len(a), len(b)
    out = []
    for i in range(out_len):
        terms = []
        if i < la:
            terms.append(a[i])
        if i < lb:
            terms.append(b[i])
        for j in range(i):
            kk = i - 1 - j
            if j < la and kk < lb:
                terms.append(jnp.minimum(a[j], b[kk]))
        while len(terms) > 1:
            terms = [jnp.maximum(terms[t], terms[t + 1])
                     for t in range(0, len(terms) - 1, 2)] + (
                         [terms[-1]] if len(terms) % 2 else [])
        out.append(terms[0])
    return out


def _tree_topk(state, k):
    w = state[0].shape[1]
    while w > 1:
        half = w // 2
        a = [s[:, :half] for s in state]
        b = [s[:, half:] for s in state]
        state = _merge_sorted(a, b, min(2 * len(state), k))
        w = half
    return jnp.concatenate(state, axis=1)


def _phase1_body(x_ref, out_ref):
    v = x_ref.shape[1]
    nfull = v // W
    rem = v % W

    def load(c):
        return x_ref[:, pl.ds(c * W, W)]

    def load_rem():
        xr = x_ref[:, pl.ds(nfull * W, rem)]
        pad = jnp.full((BLK_B, W - rem), NEG, jnp.float32)
        return jnp.concatenate([xr, pad], axis=1)

    state0 = [jnp.full((BLK_B, W), NEG, jnp.float32) for _ in range(S_FAST)]
    ngroup = nfull // 4

    def sort4(a, b, c, d):
        p = [jnp.maximum(a, b), jnp.minimum(a, b)]
        q = [jnp.maximum(c, d), jnp.minimum(c, d)]
        return _merge_sorted(p, q, 4)

    zero = jnp.zeros((BLK_B, W), jnp.float32)

    def group_step(state, s_acc, t_acc, xs):
        g = sort4(*xs)
        state = _merge_sorted(state, g, S_FAST)
        es = [jnp.exp(x) for x in xs]
        s_acc = s_acc + ((es[0] + es[1]) + (es[2] + es[3]))
        t_acc = t_acc + ((es[0] * xs[0] + es[1] * xs[1])
                         + (es[2] * xs[2] + es[3] * xs[3]))
        return state, s_acc, t_acc

    UNROLL = 8

    def body1(t, carry):
        state, s_acc, t_acc = carry
        for half in range(UNROLL):
            xs = [load(4 * (UNROLL * t + half) + u) for u in range(4)]
            state, s_acc, t_acc = group_step(state, s_acc, t_acc, xs)
        return state, s_acc, t_acc

    state, s_acc, t_acc = jax.lax.fori_loop(0, ngroup // UNROLL, body1,
                                            (state0, zero, zero))
    for t8 in range(ngroup // UNROLL * UNROLL, ngroup):
        xs = [load(4 * t8 + u) for u in range(4)]
        state, s_acc, t_acc = group_step(state, s_acc, t_acc, xs)
    tail = [load(c) for c in range(4 * ngroup, nfull)]
    if rem:
        tail.append(load_rem())
    for x in tail:
        state = _merge_sorted(state, [x], S_FAST)
        e = jnp.exp(x)
        s_acc = s_acc + e
        t_acc = t_acc + e * x

    m = jnp.max(state[0], axis=1, keepdims=True)

    topk = _tree_topk(state, K)
    tau = topk[:, K - 1:K]
    bad = jnp.any(state[S_FAST - 1] >= tau)
    bad = jnp.logical_or(bad, jnp.any(jnp.abs(m) > 60.0))

    s = jnp.sum(s_acc, axis=1, keepdims=True)
    t = jnp.sum(t_acc, axis=1, keepdims=True)
    lse = jnp.log(s)
    entropy = lse - t / s

    out_ref[...] = jnp.concatenate([topk, lse, entropy], axis=1)

    @pl.when(bad)
    def _exact_fallback():
        st0 = [jnp.full((BLK_B, W), NEG, jnp.float32) for _ in range(K)]
        st = jax.lax.fori_loop(
            0, nfull, lambda c, s: _insert_topk(s, load(c)), st0)
        if rem:
            st = _insert_topk(st, load_rem())
        topk_x = _tree_topk(st, K)
        mx = topk_x[:, 0:1]

        def body2(c, carry):
            s_acc, t_acc = carry
            x = load(c)
            e = jnp.exp(x - mx)
            return s_acc + e, t_acc + e * x

        s_acc, t_acc = jax.lax.fori_loop(0, nfull, body2, (zero, zero))
        if rem:
            xr = load_rem()
            e = jnp.exp(xr - mx)
            s_acc, t_acc = s_acc + e, t_acc + e * xr
        sx = jnp.sum(s_acc, axis=1, keepdims=True)
        tx = jnp.sum(t_acc, axis=1, keepdims=True)
        lse_x = mx + jnp.log(sx)
        ent_x = lse_x - (tx / sx)
        out_ref[...] = jnp.concatenate([topk_x, lse_x, ent_x], axis=1)


def _bn(x, g, b):
    mu = jnp.mean(x, axis=0, keepdims=True)
    d = x - mu
    var = jnp.mean(d * d, axis=0, keepdims=True)
    return g * d * jax.lax.rsqrt(var + 1e-5) + b


def _phase2_body(stats_ref, bn1_g_ref, bn1_b_ref, w1_ref, b1_ref,
                 bn2_g_ref, bn2_b_ref, w2_ref, b2_ref,
                 bn3_g_ref, bn3_b_ref, w3_ref, b3_ref, out_ref):
    stats = stats_ref[...]
    topk_l = stats[:, 0:K]
    lse = stats[:, K:K + 1]
    entropy = stats[:, K + 1:K + 2]
    p = jnp.exp(topk_l - lse)

    feats = [entropy]
    for i in range(K):
        feats.append(p[:, i:i + 1] - p)
    x = jnp.concatenate(feats, axis=1)

    h = _bn(x, bn1_g_ref[...], bn1_b_ref[...])
    h = jax.lax.dot_general(h, w1_ref[...], (((1,), (1,)), ((), ())),
                            preferred_element_type=jnp.float32) + b1_ref[...]
    h = _bn(h, bn2_g_ref[...], bn2_b_ref[...])
    h = jnp.maximum(h, 0.0)
    h = jax.lax.dot_general(h, w2_ref[...], (((1,), (1,)), ((), ())),
                            preferred_element_type=jnp.float32) + b2_ref[...]
    h = _bn(h, bn3_g_ref[...], bn3_b_ref[...])
    out_ref[...] = jax.lax.dot_general(
        h, w3_ref[...], (((1,), (1,)), ((), ())),
        preferred_element_type=jnp.float32) + b3_ref[...]


@jax.jit
def kernel(logits, ft, bn1_g, bn1_b, W1, b1, bn2_g, bn2_b, W2, b2,
           bn3_g, bn3_b, W3, b3):
    del ft
    b, v = logits.shape

    stats = pl.pallas_call(
        _phase1_body,
        grid=(b // BLK_B,),
        in_specs=[pl.BlockSpec((BLK_B, v), lambda i: (i, 0))],
        out_specs=pl.BlockSpec((BLK_B, K + 2), lambda i: (i, 0)),
        out_shape=jax.ShapeDtypeStruct((b, K + 2), jnp.float32),
        compiler_params=pltpu.CompilerParams(
            dimension_semantics=("parallel",)),
    )(logits)

    row = lambda a: a.reshape(1, -1)
    gate = pl.pallas_call(
        _phase2_body,
        out_shape=jax.ShapeDtypeStruct((b, 2), jnp.float32),
    )(stats, row(bn1_g), row(bn1_b), W1, row(b1),
      row(bn2_g), row(bn2_b), W2, row(b2),
      row(bn3_g), row(bn3_b), W3, row(b3))
    return gate
